# baseline (speedup 1.0000x reference)
import functools

import jax
import jax.numpy as jnp
from jax import lax
from jax.experimental import pallas as pl
from jax.experimental.pallas import tpu as pltpu
from jax.experimental.pallas import tpu_sc as plsc

B = 16384
F = 32
NC = 2
NS = 16
L = 16
NW = NC * NS
BPW = B // NW


def _mf_dot_body(uid_hbm, iid_hbm, ut_hbm, it_hbm, out_hbm,
                 uidx_v, iidx_v, ucols_v, icols_v, out_v, sem):
  wid = lax.axis_index("s") * NC + lax.axis_index("c")
  base = wid * BPW

  pltpu.sync_copy(uid_hbm.at[pl.ds(base, BPW)], uidx_v)
  pltpu.sync_copy(iid_hbm.at[pl.ds(base, BPW)], iidx_v)

  copies = []
  for f in range(F):
    copies.append(pltpu.async_copy(ut_hbm.at[f].at[uidx_v], ucols_v.at[f], sem))
    copies.append(pltpu.async_copy(it_hbm.at[f].at[iidx_v], icols_v.at[f], sem))
  for c in copies:
    c.wait()

  def body(g, _):
    sl = pl.ds(g * L, L)
    acc = jnp.zeros((L,), jnp.float32)
    for f in range(F):
      acc = acc + ucols_v[f, sl] * icols_v[f, sl]
    out_v[sl] = acc
    return 0

  lax.fori_loop(0, BPW // L, body, 0)

  pltpu.sync_copy(out_v, out_hbm.at[pl.ds(base, BPW)])


_mf_dot = functools.partial(
    pl.kernel,
    out_type=jax.ShapeDtypeStruct((B,), jnp.float32),
    mesh=plsc.VectorSubcoreMesh(core_axis_name="c", subcore_axis_name="s"),
    scratch_types=[
        pltpu.VMEM((BPW,), jnp.int32),
        pltpu.VMEM((BPW,), jnp.int32),
        pltpu.VMEM((F, BPW), jnp.float32),
        pltpu.VMEM((F, BPW), jnp.float32),
        pltpu.VMEM((BPW,), jnp.float32),
        pltpu.SemaphoreType.DMA,
    ],
    compiler_params=pltpu.CompilerParams(
        needs_layout_passes=False, use_tc_tiling_on_sc=False),
)(_mf_dot_body)


@jax.jit
def kernel(user_ids, item_ids, user_table, item_table):
  return _mf_dot(user_ids.astype(jnp.int32), item_ids.astype(jnp.int32),
                 user_table.T, item_table.T)


# speedup vs baseline: 5.6942x; 5.6942x over previous
import functools

import jax
import jax.numpy as jnp
from jax import lax
from jax.experimental import pallas as pl
from jax.experimental.pallas import tpu as pltpu
from jax.experimental.pallas import tpu_sc as plsc

B = 16384
F = 32
N = 1000000
NC = 2
NS = 16
L = 16
NW = NC * NS
BPW = B // NW
CH = 128
NCH = BPW // CH


def _mf_dot_body(uid_hbm, iid_hbm, ut_hbm, it_hbm, out_hbm,
                 uidx_v, iidx_v, urows_v, irows_v, out_v, sem):
  wid = lax.axis_index("s") * NC + lax.axis_index("c")
  base = wid * BPW

  pltpu.sync_copy(uid_hbm.at[pl.ds(base, BPW)], uidx_v)
  pltpu.sync_copy(iid_hbm.at[pl.ds(base, BPW)], iidx_v)

  copies = []
  for k in range(NCH):
    isl = pl.ds(k * CH, CH)
    copies.append(pltpu.async_copy(
        ut_hbm.at[uidx_v.at[isl]], urows_v.at[isl], sem))
    copies.append(pltpu.async_copy(
        it_hbm.at[iidx_v.at[isl]], irows_v.at[isl], sem))
  for c in copies:
    c.wait()

  iota = lax.iota(jnp.int32, L)
  m_last = iota == (L - 1)

  def body(g, _):
    u0 = urows_v[g, pl.ds(0, L)]
    u1 = urows_v[g, pl.ds(L, L)]
    v0 = irows_v[g, pl.ds(0, L)]
    v1 = irows_v[g, pl.ds(L, L)]
    p = u0 * v0 + u1 * v1
    s = jnp.cumsum(p)
    plsc.store_scatter(out_v, [iota * 0 + g], s, mask=m_last)
    return 0

  lax.fori_loop(0, BPW, body, 0)

  pltpu.sync_copy(out_v, out_hbm.at[pl.ds(base, BPW)])


_mf_dot = functools.partial(
    pl.kernel,
    out_type=jax.ShapeDtypeStruct((B,), jnp.float32),
    mesh=plsc.VectorSubcoreMesh(core_axis_name="c", subcore_axis_name="s"),
    scratch_types=[
        pltpu.VMEM((BPW,), jnp.int32),
        pltpu.VMEM((BPW,), jnp.int32),
        pltpu.VMEM((BPW, F), jnp.float32),
        pltpu.VMEM((BPW, F), jnp.float32),
        pltpu.VMEM((BPW,), jnp.float32),
        pltpu.SemaphoreType.DMA,
    ],
    compiler_params=pltpu.CompilerParams(
        needs_layout_passes=False, use_tc_tiling_on_sc=False),
)(_mf_dot_body)


@jax.jit
def kernel(user_ids, item_ids, user_table, item_table):
  return _mf_dot(user_ids.astype(jnp.int32), item_ids.astype(jnp.int32),
                 user_table, item_table)


# speedup vs baseline: 5.7012x; 1.0012x over previous
import functools

import jax
import jax.numpy as jnp
from jax import lax
from jax.experimental import pallas as pl
from jax.experimental.pallas import tpu as pltpu
from jax.experimental.pallas import tpu_sc as plsc

B = 16384
F = 32
N = 1000000
NC = 2
NS = 16
L = 16
NW = NC * NS
BPW = B // NW
HALF = BPW // 2
CH = 128
RPL = 4
LINES = N // RPL
LW = RPL * F


def _mf_dot_body(uid_hbm, iid_hbm, ut_hbm, it_hbm, out_hbm,
                 uidx_v, iidx_v, ulin_v, ilin_v,
                 urows_v, irows_v, out_v, sem):
  wid = lax.axis_index("s") * NC + lax.axis_index("c")
  base = wid * BPW

  pltpu.sync_copy(uid_hbm.at[pl.ds(base, BPW)], uidx_v)
  pltpu.sync_copy(iid_hbm.at[pl.ds(base, BPW)], iidx_v)

  def mkline(g, _):
    sl = pl.ds(g * L, L)
    ulin_v[sl] = lax.shift_right_logical(uidx_v[sl], 2)
    ilin_v[sl] = lax.shift_right_logical(iidx_v[sl], 2)
    return 0

  lax.fori_loop(0, BPW // L, mkline, 0)

  iota = lax.iota(jnp.int32, L)
  m_last = iota == (L - 1)

  for h in range(BPW // HALF):
    copies = []
    for j in range(HALF // CH):
      isl = pl.ds(h * HALF + j * CH, CH)
      dsl = pl.ds(j * CH, CH)
      copies.append(pltpu.async_copy(
          ut_hbm.at[ulin_v.at[isl]], urows_v.at[dsl], sem))
      copies.append(pltpu.async_copy(
          it_hbm.at[ilin_v.at[isl]], irows_v.at[dsl], sem))
    for c in copies:
      c.wait()

    def body(g16, _):
      id0 = h * HALF + g16 * L
      ucolv = (uidx_v[pl.ds(id0, L)] & 3) * F
      icolv = (iidx_v[pl.ds(id0, L)] & 3) * F
      acc = jnp.zeros((L,), jnp.float32)
      for j in range(L):
        r = g16 * L + j
        ucol = ucolv[j]
        icol = icolv[j]
        u0 = urows_v[r, pl.ds(ucol, L)]
        u1 = urows_v[r, pl.ds(ucol + L, L)]
        v0 = irows_v[r, pl.ds(icol, L)]
        v1 = irows_v[r, pl.ds(icol + L, L)]
        p = u0 * v0 + u1 * v1
        s = jnp.cumsum(p)
        acc = jnp.where(iota == j, s[L - 1], acc)
      out_v[pl.ds(id0, L)] = acc
      return 0

    lax.fori_loop(0, HALF // L, body, 0)

  pltpu.sync_copy(out_v, out_hbm.at[pl.ds(base, BPW)])


_mf_dot = functools.partial(
    pl.kernel,
    out_type=jax.ShapeDtypeStruct((B,), jnp.float32),
    mesh=plsc.VectorSubcoreMesh(core_axis_name="c", subcore_axis_name="s"),
    scratch_types=[
        pltpu.VMEM((BPW,), jnp.int32),
        pltpu.VMEM((BPW,), jnp.int32),
        pltpu.VMEM((BPW,), jnp.int32),
        pltpu.VMEM((BPW,), jnp.int32),
        pltpu.VMEM((HALF, LW), jnp.float32),
        pltpu.VMEM((HALF, LW), jnp.float32),
        pltpu.VMEM((BPW,), jnp.float32),
        pltpu.SemaphoreType.DMA,
    ],
    compiler_params=pltpu.CompilerParams(
        needs_layout_passes=False, use_tc_tiling_on_sc=True),
)(_mf_dot_body)


@jax.jit
def kernel(user_ids, item_ids, user_table, item_table):
  return _mf_dot(user_ids.astype(jnp.int32), item_ids.astype(jnp.int32),
                 user_table.reshape(LINES, LW), item_table.reshape(LINES, LW))
